# Initial kernel scaffold; baseline (speedup 1.0000x reference)
import jax
import jax.numpy as jnp
from jax.experimental import pallas as pl


def kernel(features, edge_index, W, b):
    raise NotImplementedError("write your pallas kernel here")


# speedup vs baseline: 3.6684x; 3.6684x over previous
import functools

import jax
import jax.numpy as jnp
from jax import lax
from jax.experimental import pallas as pl
from jax.experimental.pallas import tpu as pltpu
from jax.experimental.pallas import tpu_sc as plsc

N = 10000
F = 128
O = 64
NC, NS, L = 2, 16, 16
NW = NC * NS
CH = 128
NROWS = 10112
RPT = NROWS // NS
TRASH = N
HROWS = 128

_mesh = plsc.VectorSubcoreMesh(
    core_axis_name="c", subcore_axis_name="s", num_cores=NC, num_subcores=NS
)


def _wid_rows():
    cid = lax.axis_index("c")
    sid = lax.axis_index("s")
    return cid, sid * NC + cid, sid * RPT


def _deg_body(nchunk, src_hbm, dst_hbm, zflat_hbm, dstm_out, deg_out,
              src_v, dst_v, dstm_v, hist_v):
    cid = lax.axis_index("c")
    sid = lax.axis_index("s")
    w = sid * NC + cid
    pltpu.sync_copy(zflat_hbm, hist_v)
    ones = jnp.full((L,), 1.0, jnp.float32)

    def chunk(j, carry):
        pltpu.sync_copy(src_hbm.at[w, j], src_v)
        pltpu.sync_copy(dst_hbm.at[w, j], dst_v)
        for i in range(CH // L):
            s = src_v[pl.ds(i * L, L)]
            d = dst_v[pl.ds(i * L, L)]
            dm = jnp.where(s == d, TRASH, d)
            dstm_v[pl.ds(i * L, L)] = dm
            plsc.addupdate_scatter(hist_v, [dm], ones)
        pltpu.sync_copy(dstm_v, dstm_out.at[w, j])
        return carry

    lax.fori_loop(0, nchunk, chunk, 0)
    pltpu.sync_copy(hist_v, deg_out.at[w])


def _prop_body(nchunk, g_hbm, src_hbm, dstm_hbm, zrows_hbm, agg_out,
               sidx_v, didx_v, rows_v, agg_sh):
    cid, w, row0 = _wid_rows()
    pltpu.sync_copy(zrows_hbm.at[pl.ds(row0, RPT)], agg_sh.at[pl.ds(row0, RPT)])
    plsc.subcore_barrier()

    def chunk(j, carry):
        pltpu.sync_copy(src_hbm.at[w, j], sidx_v)
        pltpu.sync_copy(dstm_hbm.at[w, j], didx_v)
        pltpu.sync_copy(g_hbm.at[sidx_v], rows_v)
        pltpu.sync_copy(rows_v, agg_sh.at[didx_v], add=True)
        return carry

    lax.fori_loop(0, nchunk, chunk, 0)
    plsc.subcore_barrier()
    pltpu.sync_copy(agg_sh.at[pl.ds(row0, RPT)], agg_out.at[cid, pl.ds(row0, RPT)])


def _make_deg_kernel(nchunk):
    return functools.partial(
        pl.kernel,
        out_type=(
            jax.ShapeDtypeStruct((NW, nchunk, CH), jnp.int32),
            jax.ShapeDtypeStruct((NW, HROWS * CH), jnp.float32),
        ),
        mesh=_mesh,
        scratch_types=[
            pltpu.VMEM((CH,), jnp.int32),
            pltpu.VMEM((CH,), jnp.int32),
            pltpu.VMEM((CH,), jnp.int32),
            pltpu.VMEM((HROWS * CH,), jnp.float32),
        ],
        compiler_params=pltpu.CompilerParams(needs_layout_passes=False),
    )(functools.partial(_deg_body, nchunk))


def _make_prop_kernel(nchunk):
    return functools.partial(
        pl.kernel,
        out_type=jax.ShapeDtypeStruct((NC, NROWS, F), jnp.float32),
        mesh=_mesh,
        scratch_types=[
            pltpu.VMEM((CH,), jnp.int32),
            pltpu.VMEM((CH,), jnp.int32),
            pltpu.VMEM((CH, F), jnp.float32),
            pltpu.VMEM_SHARED((NROWS, F), jnp.float32),
        ],
    )(functools.partial(_prop_body, nchunk))


def _tc_scale_body(deg_ref, feat_ref, g_out):
    deg = jnp.sum(deg_ref[...], axis=1, keepdims=True) + 1.0
    g_out[...] = feat_ref[...] * lax.rsqrt(deg)


def _tc_mid_body(deg_ref, agg_ref, g_ref, out_ref):
    inv_deg = 1.0 / (jnp.sum(deg_ref[...], axis=1, keepdims=True) + 1.0)
    a = agg_ref[...]
    out_ref[...] = (a[0, 0:N] + a[1, 0:N] + g_ref[...]) * inv_deg


def _tc_final_body(deg_ref, agg_ref, g_ref, wt_ref, b_ref, out_ref):
    norm = lax.rsqrt(jnp.sum(deg_ref[...], axis=1, keepdims=True) + 1.0)
    a = agg_ref[...]
    h = (a[0, 0:N] + a[1, 0:N] + g_ref[...]) * norm
    out_ref[...] = (
        jnp.dot(h, wt_ref[...], preferred_element_type=jnp.float32) + b_ref[...]
    )


@jax.jit
def kernel(features, edge_index, W, b):
    e = edge_index.shape[1]
    nchunk = -(-e // (NW * CH))
    epad = NW * nchunk * CH
    src = jnp.concatenate([edge_index[0], jnp.zeros((epad - e,), jnp.int32)])
    dst = jnp.concatenate([edge_index[1], jnp.zeros((epad - e,), jnp.int32)])
    src3 = src.reshape(NW, nchunk, CH)
    dst3 = dst.reshape(NW, nchunk, CH)
    zflat = jnp.zeros((HROWS * CH,), jnp.float32)
    zrows = jnp.zeros((NROWS, F), jnp.float32)

    dstm3, deg32 = _make_deg_kernel(nchunk)(src3, dst3, zflat)
    degp = deg32[:, :N].T

    g1 = pl.pallas_call(
        _tc_scale_body,
        out_shape=jax.ShapeDtypeStruct((N, F), jnp.float32),
    )(degp, features)

    prop = _make_prop_kernel(nchunk)
    agg1 = prop(g1, src3, dstm3, zrows)

    g2 = pl.pallas_call(
        _tc_mid_body,
        out_shape=jax.ShapeDtypeStruct((N, F), jnp.float32),
    )(degp, agg1, g1)

    agg2 = prop(g2, src3, dstm3, zrows)

    out = pl.pallas_call(
        _tc_final_body,
        out_shape=jax.ShapeDtypeStruct((N, O), jnp.float32),
    )(degp, agg2, g2, W.T, b.reshape(1, O))
    return out
